# Initial kernel scaffold; baseline (speedup 1.0000x reference)
import jax
import jax.numpy as jnp
from jax.experimental import pallas as pl


def kernel(x, edge_index, edge_attr, batch, params):
    raise NotImplementedError("write your pallas kernel here")


# speedup vs baseline: 9.0569x; 9.0569x over previous
import functools
import jax
import jax.numpy as jnp
from jax import lax
from jax.experimental import pallas as pl
from jax.experimental.pallas import tpu as pltpu
from jax.experimental.pallas import tpu_sc as plsc

N = 10000
NP = 10240
E = 160000
G = 64
NC = 2
NS = 16
NW = NC * NS
EPW = E // NW
BB = 40
NSUB = EPW // BB
RPT = NP // NS

_mesh = functools.partial(plsc.VectorSubcoreMesh, core_axis_name="c", subcore_axis_name="s")


def _mm(a, b):
    return jnp.dot(a, b, preferred_element_type=jnp.float32)


def _sc_gather2(tab_a, idx_a3, tab_b, idx_b3):
    da = tab_a.shape[1]
    db = tab_b.shape[1]

    @functools.partial(
        pl.kernel,
        mesh=_mesh(),
        out_type=(
            jax.ShapeDtypeStruct((E, da), jnp.float32),
            jax.ShapeDtypeStruct((E, db), jnp.float32),
        ),
        scratch_types=[
            pltpu.VMEM((NSUB, BB), jnp.int32),
            pltpu.VMEM((BB, da), jnp.float32),
            pltpu.VMEM((NSUB, BB), jnp.int32),
            pltpu.VMEM((BB, db), jnp.float32),
            pltpu.SemaphoreType.DMA,
        ],
    )
    def k(ta, ia, tb, ib, oa, ob, ia_v, ra_v, ib_v, rb_v, sem):
        wid = lax.axis_index("s") * NC + lax.axis_index("c")
        base = wid * EPW
        pltpu.sync_copy(ia.at[wid], ia_v)
        pltpu.sync_copy(ib.at[wid], ib_v)

        def body(j, carry):
            ca = pltpu.async_copy(ta.at[ia_v.at[j]], ra_v, sem)
            cb = pltpu.async_copy(tb.at[ib_v.at[j]], rb_v, sem)
            ca.wait()
            cb.wait()
            pltpu.sync_copy(ra_v, oa.at[pl.ds(base + j * BB, BB)])
            pltpu.sync_copy(rb_v, ob.at[pl.ds(base + j * BB, BB)])
            return carry

        lax.fori_loop(0, NSUB, body, 0)

    return k(tab_a, idx_a3, tab_b, idx_b3)


def _sc_scatter_dual(ex, val, idx3, zeros_16, zeros_nd):
    d = val.shape[1]
    dc = 128
    nchunk = d // dc

    @functools.partial(
        pl.kernel,
        mesh=_mesh(),
        out_type=(
            jax.ShapeDtypeStruct((NC, NP, 16), jnp.float32),
            jax.ShapeDtypeStruct((NC, NP, d), jnp.float32),
        ),
        scratch_types=[
            pltpu.VMEM((NSUB, BB), jnp.int32),
            pltpu.VMEM((BB, 16), jnp.float32),
            pltpu.VMEM((BB, dc), jnp.float32),
            pltpu.VMEM_SHARED((NP, 16), jnp.float32),
            pltpu.VMEM_SHARED((NP, dc), jnp.float32),
        ],
    )
    def k(exh, vh, ih, z16, z, oe, ov, i_v, e_v, v_v, acc_e, acc_v):
        cid = lax.axis_index("c")
        sid = lax.axis_index("s")
        wid = sid * NC + cid
        base = wid * EPW
        pltpu.sync_copy(ih.at[wid], i_v)

        pltpu.sync_copy(z16.at[pl.ds(sid * RPT, RPT)],
                        acc_e.at[pl.ds(sid * RPT, RPT)])
        plsc.subcore_barrier()

        def ebody(j, carry):
            pltpu.sync_copy(exh.at[pl.ds(base + j * BB, BB)], e_v)
            pltpu.sync_copy(e_v, acc_e.at[i_v.at[j]], add=True)
            return carry

        lax.fori_loop(0, NSUB, ebody, 0)
        plsc.subcore_barrier()
        pltpu.sync_copy(acc_e.at[pl.ds(sid * RPT, RPT)],
                        oe.at[cid, pl.ds(sid * RPT, RPT)])
        plsc.subcore_barrier()

        for c in range(nchunk):
            pltpu.sync_copy(z.at[pl.ds(sid * RPT, RPT)],
                            acc_v.at[pl.ds(sid * RPT, RPT)])
            plsc.subcore_barrier()

            def vbody(j, carry):
                pltpu.sync_copy(vh.at[pl.ds(base + j * BB, BB), pl.ds(c * dc, dc)], v_v)
                pltpu.sync_copy(v_v, acc_v.at[i_v.at[j]], add=True)
                return carry

            lax.fori_loop(0, NSUB, vbody, 0)
            plsc.subcore_barrier()
            pltpu.sync_copy(acc_v.at[pl.ds(sid * RPT, RPT)],
                            ov.at[cid, pl.ds(sid * RPT, RPT), pl.ds(c * dc, dc)])
            plsc.subcore_barrier()

    return k(ex, val, idx3, zeros_16, zeros_nd)


def _tc_node1(x, w1, b1, w2, b2, a1s, a1d, wl, bl, wr, br):
    nb = 10
    rb = N // nb

    def body(x_r, w1_r, b1_r, w2_r, b2_r, s_r, d_r, wl_r, bl_r, wr_r, br_r,
             hs_o, hd_o, xl_o, xr_o):
        h = _mm(jax.nn.relu(_mm(x_r[...], w1_r[...]) + b1_r[...]), w2_r[...]) + b2_r[...]
        hs_o[...] = _mm(h, s_r[...])
        hd_o[...] = _mm(h, d_r[...])
        xl_o[...] = _mm(h, wl_r[...]) + bl_r[...]
        xr_o[...] = _mm(h, wr_r[...]) + br_r[...]

    full = lambda s: pl.BlockSpec(s, lambda i: (0, 0))
    return pl.pallas_call(
        body,
        grid=(nb,),
        in_specs=[
            pl.BlockSpec((rb, 128), lambda i: (i, 0)),
            full((128, 256)), full((1, 256)), full((256, 128)), full((1, 128)),
            full((128, 128)), full((128, 128)),
            full((128, 1024)), full((1, 1024)), full((128, 1024)), full((1, 1024)),
        ],
        out_specs=[
            pl.BlockSpec((rb, 128), lambda i: (i, 0)),
            pl.BlockSpec((rb, 128), lambda i: (i, 0)),
            pl.BlockSpec((rb, 1024), lambda i: (i, 0)),
            pl.BlockSpec((rb, 1024), lambda i: (i, 0)),
        ],
        out_shape=[
            jax.ShapeDtypeStruct((N, 128), jnp.float32),
            jax.ShapeDtypeStruct((N, 128), jnp.float32),
            jax.ShapeDtypeStruct((N, 1024), jnp.float32),
            jax.ShapeDtypeStruct((N, 1024), jnp.float32),
        ],
    )(x, w1, b1, w2, b2, a1s, a1d, wl, bl, wr, br)


def _tc_edge1(ea, ghs, ghd, a1e, b11, w12, b12, we):
    nb = 160
    rb = E // nb

    def body(ea_r, hs_r, hd_r, a_r, b1_r, w2_r, b2_r, we_r, e1_o, ep_o):
        pre = _mm(ea_r[...], a_r[...]) + hs_r[...] + hd_r[...] + b1_r[...]
        e1 = _mm(jax.nn.relu(pre), w2_r[...]) + b2_r[...]
        e1_o[...] = e1
        ep_o[...] = _mm(e1, we_r[...])

    full = lambda s: pl.BlockSpec(s, lambda i: (0, 0))
    return pl.pallas_call(
        body,
        grid=(nb,),
        in_specs=[
            pl.BlockSpec((rb, 16), lambda i: (i, 0)),
            pl.BlockSpec((rb, 128), lambda i: (i, 0)),
            pl.BlockSpec((rb, 128), lambda i: (i, 0)),
            full((16, 128)), full((1, 128)), full((128, 64)), full((1, 64)),
            full((64, 1024)),
        ],
        out_specs=[
            pl.BlockSpec((rb, 64), lambda i: (i, 0)),
            pl.BlockSpec((rb, 1024), lambda i: (i, 0)),
        ],
        out_shape=[
            jax.ShapeDtypeStruct((E, 64), jnp.float32),
            jax.ShapeDtypeStruct((E, 1024), jnp.float32),
        ],
    )(ea, ghs, ghd, a1e, b11, w12, b12, we)


def _tc_score(xj, xi, ep, attbd, mask16, expand):
    d = xj.shape[1]
    nb = 160
    rb = E // nb

    def body(xj_r, xi_r, ep_r, a_r, m_r, e_r, ex_o, v_o):
        s = xj_r[...] + xi_r[...] + ep_r[...]
        m = jnp.where(s >= 0, s, 0.2 * s)
        ex = jnp.exp(_mm(m, a_r[...])) * m_r[...]
        ex_o[...] = ex
        v_o[...] = xj_r[...] * _mm(ex, e_r[...])

    full = lambda s: pl.BlockSpec(s, lambda i: (0, 0))
    return pl.pallas_call(
        body,
        grid=(nb,),
        in_specs=[
            pl.BlockSpec((rb, d), lambda i: (i, 0)),
            pl.BlockSpec((rb, d), lambda i: (i, 0)),
            pl.BlockSpec((rb, d), lambda i: (i, 0)),
            full((d, 16)), full((1, 16)), full((16, d)),
        ],
        out_specs=[
            pl.BlockSpec((rb, 16), lambda i: (i, 0)),
            pl.BlockSpec((rb, d), lambda i: (i, 0)),
        ],
        out_shape=[
            jax.ShapeDtypeStruct((E, 16), jnp.float32),
            jax.ShapeDtypeStruct((E, d), jnp.float32),
        ],
    )(xj, xi, ep, attbd, mask16, expand)


def _tc_dexp(exp2, expand):
    d = expand.shape[1]
    nb = 16
    rb = NP // nb

    def body(p_r, e_r, o_r):
        dinv = 1.0 / (p_r[0] + p_r[1] + 1e-16)
        o_r[...] = _mm(dinv, e_r[...])

    return pl.pallas_call(
        body,
        grid=(nb,),
        in_specs=[
            pl.BlockSpec((2, rb, 16), lambda i: (0, i, 0)),
            pl.BlockSpec((16, d), lambda i: (0, 0)),
        ],
        out_specs=pl.BlockSpec((rb, d), lambda i: (i, 0)),
        out_shape=jax.ShapeDtypeStruct((NP, d), jnp.float32),
    )(exp2, expand)


def _tc_bn(opart, dexp, bias, g, b):
    d = opart.shape[2]
    cb = 128
    nb = d // cb

    def body(p_r, dx_r, bias_r, g_r, b_r, h_o):
        o = jax.nn.relu((p_r[0] + p_r[1]) * dx_r[...] + bias_r[...])
        rmask = lax.broadcasted_iota(jnp.int32, (NP, cb), 0) < N
        o = jnp.where(rmask, o, 0.0)
        mu = jnp.sum(o, axis=0, keepdims=True) * (1.0 / N)
        va = jnp.sum(o * o, axis=0, keepdims=True) * (1.0 / N) - mu * mu
        h_o[...] = (o - mu) / jnp.sqrt(va + 1e-5) * g_r[...] + b_r[...]

    return pl.pallas_call(
        body,
        grid=(nb,),
        in_specs=[
            pl.BlockSpec((2, NP, cb), lambda i: (0, 0, i)),
            pl.BlockSpec((NP, cb), lambda i: (0, i)),
            pl.BlockSpec((1, cb), lambda i: (0, i)),
            pl.BlockSpec((1, cb), lambda i: (0, i)),
            pl.BlockSpec((1, cb), lambda i: (0, i)),
        ],
        out_specs=pl.BlockSpec((NP, cb), lambda i: (0, i)),
        out_shape=jax.ShapeDtypeStruct((NP, d), jnp.float32),
    )(opart, dexp, bias, g, b)


def _tc_node2(h1, a2s, a2d, wl, bl, wr, br):
    nb = 16
    rb = NP // nb

    def body(h_r, s_r, d_r, wl_r, bl_r, wr_r, br_r, hs_o, hd_o, xl_o, xr_o):
        h = h_r[...]
        hs_o[...] = _mm(h, s_r[...])
        hd_o[...] = _mm(h, d_r[...])
        xl_o[...] = _mm(h, wl_r[...]) + bl_r[...]
        xr_o[...] = _mm(h, wr_r[...]) + br_r[...]

    full = lambda s: pl.BlockSpec(s, lambda i: (0, 0))
    return pl.pallas_call(
        body,
        grid=(nb,),
        in_specs=[
            pl.BlockSpec((rb, 1024), lambda i: (i, 0)),
            full((1024, 128)), full((1024, 128)),
            full((1024, 256)), full((1, 256)), full((1024, 256)), full((1, 256)),
        ],
        out_specs=[
            pl.BlockSpec((rb, 128), lambda i: (i, 0)),
            pl.BlockSpec((rb, 128), lambda i: (i, 0)),
            pl.BlockSpec((rb, 256), lambda i: (i, 0)),
            pl.BlockSpec((rb, 256), lambda i: (i, 0)),
        ],
        out_shape=[
            jax.ShapeDtypeStruct((NP, 128), jnp.float32),
            jax.ShapeDtypeStruct((NP, 128), jnp.float32),
            jax.ShapeDtypeStruct((NP, 256), jnp.float32),
            jax.ShapeDtypeStruct((NP, 256), jnp.float32),
        ],
    )(h1, a2s, a2d, wl, bl, wr, br)


def _tc_edge2(e1, ghs, ghd, a2e, b21, w22, b22, we):
    nb = 160
    rb = E // nb

    def body(e1_r, hs_r, hd_r, a_r, b1_r, w2_r, b2_r, we_r, ep_o):
        pre = _mm(e1_r[...], a_r[...]) + hs_r[...] + hd_r[...] + b1_r[...]
        e2 = _mm(jax.nn.relu(pre), w2_r[...]) + b2_r[...]
        ep_o[...] = _mm(e2, we_r[...])

    full = lambda s: pl.BlockSpec(s, lambda i: (0, 0))
    return pl.pallas_call(
        body,
        grid=(nb,),
        in_specs=[
            pl.BlockSpec((rb, 64), lambda i: (i, 0)),
            pl.BlockSpec((rb, 128), lambda i: (i, 0)),
            pl.BlockSpec((rb, 128), lambda i: (i, 0)),
            full((64, 128)), full((1, 128)), full((128, 128)), full((1, 128)),
            full((128, 256)),
        ],
        out_specs=pl.BlockSpec((rb, 256), lambda i: (i, 0)),
        out_shape=jax.ShapeDtypeStruct((E, 256), jnp.float32),
    )(e1, ghs, ghd, a2e, b21, w22, b22, we)


def _tc_final(opart, dexp, bias, g, b, batchb, fc1w, fc1b, fc2p, fc2bp):
    def body(p_r, dx_r, bias_r, g_r, b_r, bb_r, w1_r, b1_r, w2_r, b2_r, o_r):
        o = jax.nn.relu((p_r[0] + p_r[1]) * dx_r[...] + bias_r[...])
        rmask = lax.broadcasted_iota(jnp.int32, (NP, 256), 0) < N
        o = jnp.where(rmask, o, 0.0)
        mu = jnp.sum(o, axis=0, keepdims=True) * (1.0 / N)
        va = jnp.sum(o * o, axis=0, keepdims=True) * (1.0 / N) - mu * mu
        h2 = (o - mu) / jnp.sqrt(va + 1e-5) * g_r[...] + b_r[...]
        ids = bb_r[...][:, :G]
        onehot = (ids == lax.broadcasted_iota(jnp.int32, (NP, G), 1)).astype(jnp.float32)
        pooled = lax.dot_general(onehot, h2, (((0,), (0,)), ((), ())),
                                 preferred_element_type=jnp.float32)
        z = jax.nn.relu(_mm(pooled, w1_r[...]) + b1_r[...])
        o_r[...] = _mm(z, w2_r[...]) + b2_r[...]

    full2 = lambda s: pl.BlockSpec(s, lambda: tuple(0 for _ in s))
    return pl.pallas_call(
        body,
        in_specs=[
            full2((2, NP, 256)), full2((NP, 256)), full2((1, 256)), full2((1, 256)),
            full2((1, 256)), full2((NP, 128)),
            full2((256, 64)), full2((1, 64)), full2((64, 128)), full2((1, 128)),
        ],
        out_specs=full2((G, 128)),
        out_shape=jax.ShapeDtypeStruct((G, 128), jnp.float32),
    )(opart, dexp, bias, g, b, batchb, fc1w, fc1b, fc2p, fc2bp)


def _make_attbd(att, heads, c):
    cols = []
    for h in range(heads):
        cols.append(jnp.zeros((heads * c,), jnp.float32).at[c * h:c * (h + 1)].set(att[h]))
    for _ in range(16 - heads):
        cols.append(jnp.zeros((heads * c,), jnp.float32))
    return jnp.stack(cols, axis=1)


def _make_expand(heads, c):
    rows = []
    for h in range(heads):
        rows.append(jnp.zeros((heads * c,), jnp.float32).at[c * h:c * (h + 1)].set(1.0))
    for _ in range(16 - heads):
        rows.append(jnp.zeros((heads * c,), jnp.float32))
    return jnp.stack(rows, axis=0)


@jax.jit
def kernel(x, edge_index, edge_attr, batch, params):
    p = params
    src3 = edge_index[0].reshape(NW, NSUB, BB)
    dst3 = edge_index[1].reshape(NW, NSUB, BB)

    pad_c = lambda a, w: jnp.pad(a, ((0, 0), (0, w - a.shape[1])))
    pad_r = lambda a, w: jnp.pad(a, ((0, w - a.shape[0]), (0, 0)))
    a1e = pad_c(p['et1_W1'][:16], 128)
    a1s = pad_c(p['et1_W1'][16:144], 128)
    a1d = pad_c(p['et1_W1'][144:272], 128)
    w12 = pad_r(p['et1_W2'], 128)
    a2e, a2s, a2d = p['et2_W1'][:64], p['et2_W1'][64:1088], p['et2_W1'][1088:2112]
    attbd1 = _make_attbd(p['c1_att'], 4, 256)
    attbd2 = _make_attbd(p['c2_att'], 4, 64)
    expand1 = _make_expand(4, 256)
    expand2 = _make_expand(4, 64)
    mask16 = jnp.concatenate(
        [jnp.ones((4,), jnp.float32), jnp.zeros((12,), jnp.float32)]
    ).reshape(1, 16)
    zeros_16 = jnp.zeros((NP, 16), jnp.float32)
    zeros_nd = jnp.zeros((NP, 128), jnp.float32)
    r2 = lambda a: a.reshape(1, -1)

    hs1, hd1, xl1, xr1 = _tc_node1(
        x, p['nt_W1'], r2(p['nt_b1']), p['nt_W2'], r2(p['nt_b2']),
        a1s, a1d, p['c1_Wl'], r2(p['c1_bl']), p['c1_Wr'], r2(p['c1_br']))
    ghs1, ghd1 = _sc_gather2(hs1, src3, hd1, dst3)
    e1, ep1 = _tc_edge1(edge_attr, ghs1, ghd1,
                        a1e, r2(jnp.pad(p['et1_b1'], (0, 64))), w12,
                        r2(p['et1_b2']), p['c1_We'])
    xj1, xi1 = _sc_gather2(xl1, src3, xr1, dst3)
    ex1, val1 = _tc_score(xj1, xi1, ep1, attbd1, mask16, expand1)
    exn1, ovp1 = _sc_scatter_dual(ex1, val1, dst3, zeros_16, zeros_nd)
    dexp1 = _tc_dexp(exn1, expand1)
    h1 = _tc_bn(ovp1, dexp1, r2(p['c1_bias']), r2(p['bn1_g']), r2(p['bn1_b']))

    hs2, hd2, xl2, xr2 = _tc_node2(
        h1, a2s, a2d, p['c2_Wl'], r2(p['c2_bl']), p['c2_Wr'], r2(p['c2_br']))
    ghs2, ghd2 = _sc_gather2(hs2, src3, hd2, dst3)
    ep2 = _tc_edge2(e1, ghs2, ghd2,
                    a2e, r2(p['et2_b1']), p['et2_W2'], r2(p['et2_b2']),
                    p['c2_We'])
    xj2, xi2 = _sc_gather2(xl2, src3, xr2, dst3)
    ex2, val2 = _tc_score(xj2, xi2, ep2, attbd2, mask16, expand2)
    exn2, ovp2 = _sc_scatter_dual(ex2, val2, dst3, zeros_16, zeros_nd)
    dexp2 = _tc_dexp(exn2, expand2)

    batchb = jnp.broadcast_to(
        jnp.pad(batch, (0, NP - N), constant_values=G)[:, None], (NP, 128))
    fc2p = jnp.pad(p['fc2_W'], ((0, 0), (0, 127)))
    fc2bp = jnp.pad(p['fc2_b'], (0, 127)).reshape(1, 128)
    outp = _tc_final(ovp2, dexp2, r2(p['c2_bias']), r2(p['bn2_g']), r2(p['bn2_b']),
                     batchb, p['fc1_W'], r2(p['fc1_b']), fc2p, fc2bp)
    return outp[:, :1]
